# initial kernel scaffold (unmeasured)
import jax
import jax.numpy as jnp
from jax import lax
from jax.experimental import pallas as pl
from jax.experimental.pallas import tpu as pltpu

N_DEV = 16


def kernel(x, w_mat):
    m_total, _k_shard = x.shape
    _, n = w_mat.shape
    m_per = m_total // N_DEV

    def body(x_ref, w_ref, out_ref, comm_ref, send_sems, recv_sems, credit_sem):
        my = lax.axis_index("i")
        left = lax.rem(my - 1 + N_DEV, N_DEV)
        right = lax.rem(my + 1, N_DEV)

        barrier_sem = pltpu.get_barrier_semaphore()
        for nbr in [left, right]:
            pl.semaphore_signal(
                barrier_sem, inc=1,
                device_id=(nbr,), device_id_type=pl.DeviceIdType.MESH,
            )
        pl.semaphore_wait(barrier_sem, 2)

        def partial_chunk(idx):
            return jnp.dot(
                x_ref[pl.ds(idx * m_per, m_per), :],
                w_ref[...],
                preferred_element_type=jnp.float32,
            )

        idx0 = lax.rem(my - 1 + N_DEV, N_DEV)
        comm_ref[0] = partial_chunk(idx0)

        for s in range(N_DEV - 1):
            send_slot = s % 2
            recv_slot = 1 - send_slot

            if s >= 1:
                pl.semaphore_wait(credit_sem, 1)

            rdma = pltpu.make_async_remote_copy(
                src_ref=comm_ref.at[send_slot],
                dst_ref=comm_ref.at[recv_slot],
                send_sem=send_sems.at[send_slot],
                recv_sem=recv_sems.at[recv_slot],
                device_id=(right,),
                device_id_type=pl.DeviceIdType.MESH,
            )
            rdma.start()
            rdma.wait_send()
            pl.semaphore_signal(
                credit_sem, inc=1,
                device_id=(left,), device_id_type=pl.DeviceIdType.MESH,
            )
            rdma.wait_recv()

            idx = lax.rem(my - 2 - s + 2 * N_DEV, N_DEV)
            p = partial_chunk(idx)
            if s < N_DEV - 2:
                comm_ref[recv_slot] = comm_ref[recv_slot] + p
            else:
                out_ref[...] = jnp.maximum(comm_ref[recv_slot] + p, 0.0)

    return pl.pallas_call(
        body,
        out_shape=jax.ShapeDtypeStruct((m_per, n), jnp.float32),
        in_specs=[
            pl.BlockSpec(memory_space=pltpu.VMEM),
            pl.BlockSpec(memory_space=pltpu.VMEM),
        ],
        out_specs=pl.BlockSpec(memory_space=pltpu.VMEM),
        scratch_shapes=[
            pltpu.VMEM((2, m_per, n), jnp.float32),
            pltpu.SemaphoreType.DMA((2,)),
            pltpu.SemaphoreType.DMA((2,)),
            pltpu.SemaphoreType.REGULAR,
        ],
        compiler_params=pltpu.CompilerParams(collective_id=0),
    )(x, w_mat)


# baseline (device time: 1436787 ns/iter reference)
import jax
import jax.numpy as jnp
from jax import lax
from jax.experimental import pallas as pl
from jax.experimental.pallas import tpu as pltpu

N_DEV = 16


def kernel(x, w_mat):
    m_total, _k_shard = x.shape
    _, n = w_mat.shape
    m_per = m_total // N_DEV

    def body(x_ref, w_ref, out_ref, comm_ref, send_sems, recv_sems, credit_sem):
        my = lax.axis_index("i")
        left = lax.rem(my - 1 + N_DEV, N_DEV)
        right = lax.rem(my + 1, N_DEV)

        barrier_sem = pltpu.get_barrier_semaphore()
        for nbr in [left, right]:
            pl.semaphore_signal(
                barrier_sem, inc=1,
                device_id=(nbr,), device_id_type=pl.DeviceIdType.MESH,
            )
        pl.semaphore_wait(barrier_sem, 2)

        def partial_chunk(idx):
            return jnp.dot(
                x_ref[pl.ds(idx * m_per, m_per), :],
                w_ref[...],
                preferred_element_type=jnp.float32,
            )

        idx0 = lax.rem(my - 1 + N_DEV, N_DEV)
        comm_ref[0] = partial_chunk(idx0)

        for s in range(N_DEV - 1):
            send_slot = s % 2
            recv_slot = 1 - send_slot

            if s >= 1:
                pl.semaphore_wait(credit_sem, 1)

            rdma = pltpu.make_async_remote_copy(
                src_ref=comm_ref.at[send_slot],
                dst_ref=comm_ref.at[recv_slot],
                send_sem=send_sems.at[send_slot],
                recv_sem=recv_sems.at[recv_slot],
                device_id=(right,),
                device_id_type=pl.DeviceIdType.MESH,
            )
            rdma.start()
            rdma.wait_send()
            if s < N_DEV - 2:
                pl.semaphore_signal(
                    credit_sem, inc=1,
                    device_id=(left,), device_id_type=pl.DeviceIdType.MESH,
                )
            rdma.wait_recv()

            idx = lax.rem(my - 2 - s + 2 * N_DEV, N_DEV)
            p = partial_chunk(idx)
            if s < N_DEV - 2:
                comm_ref[recv_slot] = comm_ref[recv_slot] + p
            else:
                out_ref[...] = jnp.maximum(comm_ref[recv_slot] + p, 0.0)

    return pl.pallas_call(
        body,
        out_shape=jax.ShapeDtypeStruct((m_per, n), jnp.float32),
        in_specs=[
            pl.BlockSpec(memory_space=pltpu.VMEM),
            pl.BlockSpec(memory_space=pltpu.VMEM),
        ],
        out_specs=pl.BlockSpec(memory_space=pltpu.VMEM),
        scratch_shapes=[
            pltpu.VMEM((2, m_per, n), jnp.float32),
            pltpu.SemaphoreType.DMA((2,)),
            pltpu.SemaphoreType.DMA((2,)),
            pltpu.SemaphoreType.REGULAR,
        ],
        compiler_params=pltpu.CompilerParams(collective_id=0),
    )(x, w_mat)


# device time: 758292 ns/iter; 1.8948x vs baseline; 1.8948x over previous
import jax
import jax.numpy as jnp
from jax import lax
from jax.experimental import pallas as pl
from jax.experimental.pallas import tpu as pltpu

N_DEV = 16


def kernel(x, w_mat):
    m_total, _k_shard = x.shape
    _, n = w_mat.shape
    m_per = m_total // N_DEV
    half = n // 2

    def body(x_ref, w_ref, out_ref, comm_r, comm_l,
             send_r, recv_r, send_l, recv_l, credit_r, credit_l):
        my = lax.axis_index("i")
        left = lax.rem(my - 1 + N_DEV, N_DEV)
        right = lax.rem(my + 1, N_DEV)

        barrier_sem = pltpu.get_barrier_semaphore()
        for nbr in [left, right]:
            pl.semaphore_signal(
                barrier_sem, inc=1,
                device_id=(nbr,), device_id_type=pl.DeviceIdType.MESH,
            )
        pl.semaphore_wait(barrier_sem, 2)

        def partial_r(idx):
            return jnp.dot(
                x_ref[pl.ds(idx * m_per, m_per), :], w_ref[:, :half],
                preferred_element_type=jnp.float32,
            )

        def partial_l(idx):
            return jnp.dot(
                x_ref[pl.ds(idx * m_per, m_per), :], w_ref[:, half:],
                preferred_element_type=jnp.float32,
            )

        comm_r[0] = partial_r(lax.rem(my - 1 + N_DEV, N_DEV))
        comm_l[0] = partial_l(lax.rem(my + 1, N_DEV))

        for s in range(N_DEV - 1):
            ss = s % 2
            rs = 1 - ss

            if s >= 1:
                pl.semaphore_wait(credit_r, 1)
                pl.semaphore_wait(credit_l, 1)

            rdma_r = pltpu.make_async_remote_copy(
                src_ref=comm_r.at[ss], dst_ref=comm_r.at[rs],
                send_sem=send_r.at[ss], recv_sem=recv_r.at[rs],
                device_id=(right,), device_id_type=pl.DeviceIdType.MESH,
            )
            rdma_l = pltpu.make_async_remote_copy(
                src_ref=comm_l.at[ss], dst_ref=comm_l.at[rs],
                send_sem=send_l.at[ss], recv_sem=recv_l.at[rs],
                device_id=(left,), device_id_type=pl.DeviceIdType.MESH,
            )
            rdma_r.start()
            rdma_l.start()

            idx_r = lax.rem(my - 2 - s + 2 * N_DEV, N_DEV)
            idx_l = lax.rem(my + 2 + s, N_DEV)
            p_r = partial_r(idx_r)
            p_l = partial_l(idx_l)

            rdma_r.wait_send()
            rdma_l.wait_send()
            if s < N_DEV - 2:
                pl.semaphore_signal(
                    credit_r, inc=1,
                    device_id=(left,), device_id_type=pl.DeviceIdType.MESH,
                )
                pl.semaphore_signal(
                    credit_l, inc=1,
                    device_id=(right,), device_id_type=pl.DeviceIdType.MESH,
                )

            rdma_r.wait_recv()
            rdma_l.wait_recv()
            if s < N_DEV - 2:
                comm_r[rs] = comm_r[rs] + p_r
                comm_l[rs] = comm_l[rs] + p_l
            else:
                out_ref[:, :half] = jnp.maximum(comm_r[rs] + p_r, 0.0)
                out_ref[:, half:] = jnp.maximum(comm_l[rs] + p_l, 0.0)

    return pl.pallas_call(
        body,
        out_shape=jax.ShapeDtypeStruct((m_per, n), jnp.float32),
        in_specs=[
            pl.BlockSpec(memory_space=pltpu.VMEM),
            pl.BlockSpec(memory_space=pltpu.VMEM),
        ],
        out_specs=pl.BlockSpec(memory_space=pltpu.VMEM),
        scratch_shapes=[
            pltpu.VMEM((2, m_per, half), jnp.float32),
            pltpu.VMEM((2, m_per, half), jnp.float32),
            pltpu.SemaphoreType.DMA((2,)),
            pltpu.SemaphoreType.DMA((2,)),
            pltpu.SemaphoreType.DMA((2,)),
            pltpu.SemaphoreType.DMA((2,)),
            pltpu.SemaphoreType.REGULAR,
            pltpu.SemaphoreType.REGULAR,
        ],
        compiler_params=pltpu.CompilerParams(
            collective_id=0, vmem_limit_bytes=100 * 1024 * 1024,
        ),
    )(x, w_mat)


# device time: 701374 ns/iter; 2.0485x vs baseline; 1.0812x over previous
import jax
import jax.numpy as jnp
from jax import lax
from jax.experimental import pallas as pl
from jax.experimental.pallas import tpu as pltpu

N_DEV = 16
N_RINGS = 4
N_HOPS = N_DEV - 1
N_SLOTS = 3


def kernel(x, w_mat):
    m_total, _k_shard = x.shape
    _, n = w_mat.shape
    m_per = m_total // N_DEV
    q = n // N_RINGS

    def body(x_ref, w_ref, out_ref, comm, send_sems, recv_sems, credit_sems):
        my = lax.axis_index("i")
        left = lax.rem(my - 1 + N_DEV, N_DEV)
        right = lax.rem(my + 1, N_DEV)

        def idx_right(k):
            return lax.rem(my - 2 - k + 2 * N_DEV, N_DEV)

        def idx_left(k):
            return lax.rem(my + 2 + k, N_DEV)

        rings = {
            0: (right, left, idx_right),
            1: (right, left, idx_right),
            2: (left, right, idx_left),
            3: (left, right, idx_left),
        }
        ORDER = (0, 2, 1, 3)

        barrier_sem = pltpu.get_barrier_semaphore()
        for nbr in [left, right]:
            pl.semaphore_signal(
                barrier_sem, inc=1,
                device_id=(nbr,), device_id_type=pl.DeviceIdType.MESH,
            )
        pl.semaphore_wait(barrier_sem, 2)

        def partial(g, idx):
            return jnp.dot(
                x_ref[pl.ds(idx * m_per, m_per), :],
                w_ref[:, g * q:(g + 1) * q],
                preferred_element_type=jnp.float32,
            )

        def make_rdma(g, k):
            return pltpu.make_async_remote_copy(
                src_ref=comm.at[g, k % N_SLOTS],
                dst_ref=comm.at[g, (k + 1) % N_SLOTS],
                send_sem=send_sems.at[g, k % N_SLOTS],
                recv_sem=recv_sems.at[g, (k + 1) % N_SLOTS],
                device_id=(rings[g][0],),
                device_id_type=pl.DeviceIdType.MESH,
            )

        rdmas = {}

        for g in ORDER:
            seed_idx = left if g < 2 else right
            comm[g, 0] = partial(g, seed_idx)
            rdmas[(g, 0)] = make_rdma(g, 0)
            rdmas[(g, 0)].start()

        p = {}
        for g in ORDER:
            p[g] = partial(g, rings[g][2](0))

        for k in range(N_HOPS):
            rs = (k + 1) % N_SLOTS
            for g in ORDER:
                rdmas[(g, k)].wait_recv()
                if k < N_HOPS - 1:
                    comm[g, rs] = comm[g, rs] + p[g]
                    if k >= 1:
                        pl.semaphore_wait(credit_sems.at[g], 1)
                    rdmas[(g, k + 1)] = make_rdma(g, k + 1)
                    rdmas[(g, k + 1)].start()
                else:
                    out_ref[:, g * q:(g + 1) * q] = jnp.maximum(
                        comm[g, rs] + p[g], 0.0
                    )
            for g in ORDER:
                rdmas[(g, k)].wait_send()
                if k < N_HOPS - 2:
                    pl.semaphore_signal(
                        credit_sems.at[g], inc=1,
                        device_id=(rings[g][1],),
                        device_id_type=pl.DeviceIdType.MESH,
                    )
            if k < N_HOPS - 1:
                for g in ORDER:
                    p[g] = partial(g, rings[g][2](k + 1))

    return pl.pallas_call(
        body,
        out_shape=jax.ShapeDtypeStruct((m_per, n), jnp.float32),
        in_specs=[
            pl.BlockSpec(memory_space=pltpu.VMEM),
            pl.BlockSpec(memory_space=pltpu.VMEM),
        ],
        out_specs=pl.BlockSpec(memory_space=pltpu.VMEM),
        scratch_shapes=[
            pltpu.VMEM((N_RINGS, N_SLOTS, m_per, q), jnp.float32),
            pltpu.SemaphoreType.DMA((N_RINGS, N_SLOTS)),
            pltpu.SemaphoreType.DMA((N_RINGS, N_SLOTS)),
            pltpu.SemaphoreType.REGULAR((N_RINGS,)),
        ],
        compiler_params=pltpu.CompilerParams(
            collective_id=0, vmem_limit_bytes=100 * 1024 * 1024,
        ),
    )(x, w_mat)


# device time: 701341 ns/iter; 2.0486x vs baseline; 1.0000x over previous
import jax
import jax.numpy as jnp
from jax import lax
from jax.experimental import pallas as pl
from jax.experimental.pallas import tpu as pltpu

N_DEV = 16
N_RINGS = 8
N_HOPS = N_DEV - 1
N_SLOTS = 3


def kernel(x, w_mat):
    m_total, _k_shard = x.shape
    _, n = w_mat.shape
    m_per = m_total // N_DEV
    q = n // N_RINGS

    def body(x_ref, w_ref, out_ref, comm, send_sems, recv_sems, credit_sems):
        my = lax.axis_index("i")
        left = lax.rem(my - 1 + N_DEV, N_DEV)
        right = lax.rem(my + 1, N_DEV)

        def idx_right(k):
            return lax.rem(my - 2 - k + 2 * N_DEV, N_DEV)

        def idx_left(k):
            return lax.rem(my + 2 + k, N_DEV)

        rings = {
            g: (
                (right, left, idx_right) if g < N_RINGS // 2
                else (left, right, idx_left)
            )
            for g in range(N_RINGS)
        }
        ORDER = tuple(
            g for i in range(N_RINGS // 2) for g in (i, N_RINGS // 2 + i)
        )

        barrier_sem = pltpu.get_barrier_semaphore()
        for nbr in [left, right]:
            pl.semaphore_signal(
                barrier_sem, inc=1,
                device_id=(nbr,), device_id_type=pl.DeviceIdType.MESH,
            )
        pl.semaphore_wait(barrier_sem, 2)

        def partial(g, idx):
            return jnp.dot(
                x_ref[pl.ds(idx * m_per, m_per), :],
                w_ref[:, g * q:(g + 1) * q],
                preferred_element_type=jnp.float32,
            )

        def make_rdma(g, k):
            return pltpu.make_async_remote_copy(
                src_ref=comm.at[g, k % N_SLOTS],
                dst_ref=comm.at[g, (k + 1) % N_SLOTS],
                send_sem=send_sems.at[g, k % N_SLOTS],
                recv_sem=recv_sems.at[g, (k + 1) % N_SLOTS],
                device_id=(rings[g][0],),
                device_id_type=pl.DeviceIdType.MESH,
            )

        rdmas = {}

        for g in ORDER:
            seed_idx = left if g < N_RINGS // 2 else right
            comm[g, 0] = partial(g, seed_idx)
            rdmas[(g, 0)] = make_rdma(g, 0)
            rdmas[(g, 0)].start()

        p = {}
        for g in ORDER:
            p[g] = partial(g, rings[g][2](0))

        for k in range(N_HOPS):
            rs = (k + 1) % N_SLOTS
            for g in ORDER:
                rdmas[(g, k)].wait_recv()
                if k < N_HOPS - 1:
                    comm[g, rs] = comm[g, rs] + p[g]
                    if k >= 1:
                        pl.semaphore_wait(credit_sems.at[g], 1)
                    rdmas[(g, k + 1)] = make_rdma(g, k + 1)
                    rdmas[(g, k + 1)].start()
                else:
                    out_ref[:, g * q:(g + 1) * q] = jnp.maximum(
                        comm[g, rs] + p[g], 0.0
                    )
            for g in ORDER:
                rdmas[(g, k)].wait_send()
                if k < N_HOPS - 2:
                    pl.semaphore_signal(
                        credit_sems.at[g], inc=1,
                        device_id=(rings[g][1],),
                        device_id_type=pl.DeviceIdType.MESH,
                    )
            if k < N_HOPS - 1:
                for g in ORDER:
                    p[g] = partial(g, rings[g][2](k + 1))

    return pl.pallas_call(
        body,
        out_shape=jax.ShapeDtypeStruct((m_per, n), jnp.float32),
        in_specs=[
            pl.BlockSpec(memory_space=pltpu.VMEM),
            pl.BlockSpec(memory_space=pltpu.VMEM),
        ],
        out_specs=pl.BlockSpec(memory_space=pltpu.VMEM),
        scratch_shapes=[
            pltpu.VMEM((N_RINGS, N_SLOTS, m_per, q), jnp.float32),
            pltpu.SemaphoreType.DMA((N_RINGS, N_SLOTS)),
            pltpu.SemaphoreType.DMA((N_RINGS, N_SLOTS)),
            pltpu.SemaphoreType.REGULAR((N_RINGS,)),
        ],
        compiler_params=pltpu.CompilerParams(
            collective_id=0, vmem_limit_bytes=100 * 1024 * 1024,
        ),
    )(x, w_mat)
